# baseline (speedup 1.0000x reference)
import functools

import jax
import jax.numpy as jnp
from jax.experimental import pallas as pl
from jax.experimental.pallas import tpu as pltpu


def _softmax_block(logits):
    m = jnp.max(logits, axis=-1, keepdims=True)
    e = jnp.exp(logits - m)
    s = jnp.sum(e, axis=-1, keepdims=True)
    return e / s


def _gate_kernel(xa_ref, xb_ref, wt_ref, b_ref, probs_ref, ent_ref):
    block = xa_ref.shape[0]
    wt = wt_ref[...]
    bias = b_ref[...]
    pa = _softmax_block(
        jnp.dot(xa_ref[...].astype(jnp.bfloat16), wt,
                preferred_element_type=jnp.float32) + bias)
    pb = _softmax_block(
        jnp.dot(xb_ref[...].astype(jnp.bfloat16), wt,
                preferred_element_type=jnp.float32) + bias)
    probs_ref[:block, :] = pa
    probs_ref[block:, :] = pb
    plogp = pa * jnp.log(pa + 1e-9) + pb * jnp.log(pb + 1e-9)
    ent_ref[...] = jnp.sum(plogp, axis=0, keepdims=True)[None]


@functools.partial(jax.jit, static_argnames=("block",))
def _switch_gate(x, W, b, block=512):
    tokens, in_dim = x.shape
    num_experts = W.shape[0]
    wt = W.T.astype(jnp.bfloat16)
    b2 = b.reshape(1, num_experts)
    nb = tokens // (2 * block)
    probs, ent_parts = pl.pallas_call(
        _gate_kernel,
        grid=(nb,),
        in_specs=[
            pl.BlockSpec((block, in_dim), lambda i: (2 * i, 0)),
            pl.BlockSpec((block, in_dim), lambda i: (2 * i + 1, 0)),
            pl.BlockSpec((in_dim, num_experts), lambda i: (0, 0)),
            pl.BlockSpec((1, num_experts), lambda i: (0, 0)),
        ],
        out_specs=[
            pl.BlockSpec((2 * block, num_experts), lambda i: (i, 0)),
            pl.BlockSpec((1, 1, num_experts), lambda i: (i, 0, 0)),
        ],
        out_shape=[
            jax.ShapeDtypeStruct((tokens, num_experts), jnp.float32),
            jax.ShapeDtypeStruct((nb, 1, num_experts), jnp.float32),
        ],
        compiler_params=pltpu.CompilerParams(
            dimension_semantics=("parallel",),
        ),
    )(x, x, wt, b2)
    gate_entropy = -(jnp.sum(ent_parts) / tokens)
    return probs, gate_entropy


def kernel(x, W, b):
    return _switch_gate(x, W, b)


# speedup vs baseline: 1.0350x; 1.0350x over previous
import functools

import jax
import jax.numpy as jnp
from jax import lax
from jax.experimental import pallas as pl
from jax.experimental.pallas import tpu as pltpu


def _gate_kernel(x_ref, w_ref, b_ref, probs_ref, ent_ref, w_scr, acc_ref):
    i = pl.program_id(0)
    nb = pl.num_programs(0)

    @pl.when(i == 0)
    def _init():
        w_scr[...] = w_ref[...].astype(jnp.bfloat16)
        acc_ref[0] = 0.0

    x = x_ref[...].astype(jnp.bfloat16)
    logits = lax.dot_general(
        x, w_scr[...], (((1,), (1,)), ((), ())),
        preferred_element_type=jnp.float32)
    logits = logits + b_ref[...][None, :]
    m = jnp.max(logits, axis=-1, keepdims=True)
    e = jnp.exp(logits - m)
    s = jnp.sum(e, axis=-1, keepdims=True)
    p = e / s
    probs_ref[...] = p
    acc_ref[0] += jnp.sum(p * jnp.log(p + 1e-9))

    @pl.when(i == nb - 1)
    def _finalize():
        ent_ref[0] = -acc_ref[0] / (nb * x_ref.shape[0])


@functools.partial(jax.jit, static_argnames=("block",))
def _switch_gate(x, W, b, block=1024):
    tokens, in_dim = x.shape
    num_experts = W.shape[0]
    nb = tokens // block
    probs, ent = pl.pallas_call(
        _gate_kernel,
        grid=(nb,),
        in_specs=[
            pl.BlockSpec((block, in_dim), lambda i: (i, 0)),
            pl.BlockSpec((num_experts, in_dim), lambda i: (0, 0)),
            pl.BlockSpec((num_experts,), lambda i: (0,)),
        ],
        out_specs=[
            pl.BlockSpec((block, num_experts), lambda i: (i, 0)),
            pl.BlockSpec(memory_space=pltpu.SMEM),
        ],
        out_shape=[
            jax.ShapeDtypeStruct((tokens, num_experts), jnp.float32),
            jax.ShapeDtypeStruct((1,), jnp.float32),
        ],
        scratch_shapes=[
            pltpu.VMEM((num_experts, in_dim), jnp.bfloat16),
            pltpu.SMEM((1,), jnp.float32),
        ],
        compiler_params=pltpu.CompilerParams(
            dimension_semantics=("arbitrary",),
        ),
    )(x, W, b)
    return probs, ent[0]


def kernel(x, W, b):
    return _switch_gate(x, W, b)


# speedup vs baseline: 1.0468x; 1.0113x over previous
import functools

import jax
import jax.numpy as jnp
from jax import lax
from jax.experimental import pallas as pl
from jax.experimental.pallas import tpu as pltpu


def _softmax_rows(logits):
    m = jnp.max(logits, axis=-1, keepdims=True)
    e = jnp.exp(logits - m)
    s = jnp.sum(e, axis=-1, keepdims=True)
    return e / s


def _gate_kernel(xa_ref, xb_ref, w_ref, b_ref, probs_ref, ent_ref,
                 w_scr, acc_ref):
    i = pl.program_id(0)
    nb = pl.num_programs(0)
    block = xa_ref.shape[0]

    @pl.when(i == 0)
    def _init():
        w_scr[...] = w_ref[...].astype(jnp.bfloat16)
        acc_ref[0] = 0.0

    w = w_scr[...]
    bias = b_ref[...][None, :]
    pa = _softmax_rows(lax.dot_general(
        xa_ref[...].astype(jnp.bfloat16), w, (((1,), (1,)), ((), ())),
        preferred_element_type=jnp.float32) + bias)
    pb = _softmax_rows(lax.dot_general(
        xb_ref[...].astype(jnp.bfloat16), w, (((1,), (1,)), ((), ())),
        preferred_element_type=jnp.float32) + bias)
    probs_ref[:block, :] = pa
    probs_ref[block:, :] = pb
    acc_ref[0] += jnp.sum(pa * jnp.log(pa + 1e-9) + pb * jnp.log(pb + 1e-9))

    @pl.when(i == nb - 1)
    def _finalize():
        ent_ref[0] = -acc_ref[0] / (nb * 2 * block)


@functools.partial(jax.jit, static_argnames=("block",))
def _switch_gate(x, W, b, block=512):
    tokens, in_dim = x.shape
    num_experts = W.shape[0]
    nb = tokens // (2 * block)
    probs, ent = pl.pallas_call(
        _gate_kernel,
        grid=(nb,),
        in_specs=[
            pl.BlockSpec((block, in_dim), lambda i: (2 * i, 0)),
            pl.BlockSpec((block, in_dim), lambda i: (2 * i + 1, 0)),
            pl.BlockSpec((num_experts, in_dim), lambda i: (0, 0)),
            pl.BlockSpec((num_experts,), lambda i: (0,)),
        ],
        out_specs=[
            pl.BlockSpec((2 * block, num_experts), lambda i: (i, 0)),
            pl.BlockSpec(memory_space=pltpu.SMEM),
        ],
        out_shape=[
            jax.ShapeDtypeStruct((tokens, num_experts), jnp.float32),
            jax.ShapeDtypeStruct((1,), jnp.float32),
        ],
        scratch_shapes=[
            pltpu.VMEM((num_experts, in_dim), jnp.bfloat16),
            pltpu.SMEM((1,), jnp.float32),
        ],
        compiler_params=pltpu.CompilerParams(
            dimension_semantics=("arbitrary",),
        ),
    )(x, x, W, b)
    return probs, ent[0]


def kernel(x, W, b):
    return _switch_gate(x, W, b)


# speedup vs baseline: 1.0498x; 1.0029x over previous
import functools

import jax
import jax.numpy as jnp
from jax import lax
from jax.experimental import pallas as pl
from jax.experimental.pallas import tpu as pltpu

NSTREAMS = 4
BLOCK = 256


def _softmax_rows(logits):
    m = jnp.max(logits, axis=-1, keepdims=True)
    e = jnp.exp(logits - m)
    s = jnp.sum(e, axis=-1, keepdims=True)
    return e / s


def _gate_kernel(*refs):
    x_refs = refs[:NSTREAMS]
    w_ref, b_ref, probs_ref, ent_ref, w_scr, acc_ref = refs[NSTREAMS:]
    i = pl.program_id(0)
    nb = pl.num_programs(0)
    block = x_refs[0].shape[0]

    @pl.when(i == 0)
    def _init():
        w_scr[...] = w_ref[...].astype(jnp.bfloat16)
        acc_ref[0] = 0.0

    w = w_scr[...]
    bias = b_ref[...][None, :]
    total = jnp.zeros((), jnp.float32)
    for k, x_ref in enumerate(x_refs):
        p = _softmax_rows(lax.dot_general(
            x_ref[...].astype(jnp.bfloat16), w, (((1,), (1,)), ((), ())),
            preferred_element_type=jnp.float32) + bias)
        probs_ref[k * block:(k + 1) * block, :] = p
        total += jnp.sum(p * jnp.log(p + 1e-9))
    acc_ref[0] += total

    @pl.when(i == nb - 1)
    def _finalize():
        ent_ref[0] = -acc_ref[0] / (nb * NSTREAMS * block)


@jax.jit
def _switch_gate(x, W, b):
    tokens, in_dim = x.shape
    num_experts = W.shape[0]
    step_rows = NSTREAMS * BLOCK
    nb = tokens // step_rows

    def _xspec(k):
        return pl.BlockSpec((BLOCK, in_dim), lambda i, k=k: (NSTREAMS * i + k, 0))

    probs, ent = pl.pallas_call(
        _gate_kernel,
        grid=(nb,),
        in_specs=[_xspec(k) for k in range(NSTREAMS)] + [
            pl.BlockSpec((num_experts, in_dim), lambda i: (0, 0)),
            pl.BlockSpec((num_experts,), lambda i: (0,)),
        ],
        out_specs=[
            pl.BlockSpec((step_rows, num_experts), lambda i: (i, 0)),
            pl.BlockSpec(memory_space=pltpu.SMEM),
        ],
        out_shape=[
            jax.ShapeDtypeStruct((tokens, num_experts), jnp.float32),
            jax.ShapeDtypeStruct((1,), jnp.float32),
        ],
        scratch_shapes=[
            pltpu.VMEM((num_experts, in_dim), jnp.bfloat16),
            pltpu.SMEM((1,), jnp.float32),
        ],
        compiler_params=pltpu.CompilerParams(
            dimension_semantics=("arbitrary",),
        ),
    )(*([x] * NSTREAMS), W, b)
    return probs, ent[0]


def kernel(x, W, b):
    return _switch_gate(x, W, b)
